# Initial kernel scaffold; baseline (speedup 1.0000x reference)
import jax
import jax.numpy as jnp
from jax.experimental import pallas as pl


def kernel(x, table):
    raise NotImplementedError("write your pallas kernel here")


# speedup vs baseline: 4.0894x; 4.0894x over previous
import jax
import jax.numpy as jnp
from jax import lax
from jax.experimental import pallas as pl
from jax.experimental.pallas import tpu as pltpu
from jax.experimental.pallas import tpu_sc as plsc

_CHUNK = 128


def _make_lookup(n_chunks, emb_dim):
    info = plsc.get_sparse_core_info()
    nw = info.num_cores * info.num_subcores
    assert n_chunks % nw == 0
    cpw = n_chunks // nw
    mesh = plsc.VectorSubcoreMesh(core_axis_name="c", subcore_axis_name="s")

    def body(table_hbm, idx_hbm, out_hbm, idx_v, rows_v, gsem):
        wid = lax.axis_index("s") * info.num_cores + lax.axis_index("c")
        chunk0 = wid * cpw
        pltpu.sync_copy(idx_hbm.at[pl.ds(chunk0 * _CHUNK, cpw * _CHUNK)], idx_v)

        def step(j, carry):
            pltpu.async_copy(
                table_hbm.at[idx_v.at[pl.ds(j * _CHUNK, _CHUNK)]], rows_v, gsem
            ).wait()
            pltpu.sync_copy(
                rows_v, out_hbm.at[pl.ds((chunk0 + j) * _CHUNK, _CHUNK)]
            )
            return carry

        lax.fori_loop(0, cpw, step, 0)

    return pl.kernel(
        body,
        out_type=jax.ShapeDtypeStruct((n_chunks * _CHUNK, emb_dim), jnp.float32),
        mesh=mesh,
        compiler_params=pltpu.CompilerParams(use_tc_tiling_on_sc=False),
        scratch_types=[
            pltpu.VMEM((cpw * _CHUNK,), jnp.int32),
            pltpu.VMEM((_CHUNK, emb_dim), jnp.float32),
            pltpu.SemaphoreType.DMA,
        ],
    )


def kernel(x, table):
    b, s = x.shape
    n = b * s
    emb_dim = table.shape[1]
    idx_flat = x.reshape(n)
    out = _make_lookup(n // _CHUNK, emb_dim)(table, idx_flat)
    return out.reshape(b, s, emb_dim)


# speedup vs baseline: 4.6804x; 1.1445x over previous
import jax
import jax.numpy as jnp
from jax import lax
from jax.experimental import pallas as pl
from jax.experimental.pallas import tpu as pltpu
from jax.experimental.pallas import tpu_sc as plsc

_CHUNK = 128
_NBUF = 5


def _make_lookup(n_chunks, emb_dim):
    info = plsc.get_sparse_core_info()
    nw = info.num_cores * info.num_subcores
    assert n_chunks % (nw * _NBUF) == 0
    cpw = n_chunks // nw
    niter = cpw // _NBUF
    mesh = plsc.VectorSubcoreMesh(core_axis_name="c", subcore_axis_name="s")

    def body(table_hbm, idx_hbm, out_hbm, idx_v, *bufs):
        rows = bufs[:_NBUF]
        gs = bufs[_NBUF : 2 * _NBUF]
        ws = bufs[2 * _NBUF : 3 * _NBUF]
        wid = lax.axis_index("s") * info.num_cores + lax.axis_index("c")
        chunk0 = wid * cpw
        pltpu.sync_copy(idx_hbm.at[pl.ds(chunk0 * _CHUNK, cpw * _CHUNK)], idx_v)

        def gather(j, b):
            pltpu.async_copy(
                table_hbm.at[idx_v.at[pl.ds(j * _CHUNK, _CHUNK)]], rows[b], gs[b]
            )

        def wait_gather(j, b):
            pltpu.make_async_copy(
                table_hbm.at[idx_v.at[pl.ds(j * _CHUNK, _CHUNK)]], rows[b], gs[b]
            ).wait()

        def wait_wb(b):
            pltpu.make_async_copy(
                rows[b], out_hbm.at[pl.ds(0, _CHUNK)], ws[b]
            ).wait()

        for b in range(_NBUF - 1):
            gather(b, b)

        def outer(g, carry):
            for b in range(_NBUF):
                j = g * _NBUF + b
                p = (b - 1) % _NBUF
                wait_gather(j, b)
                pltpu.async_copy(
                    rows[b], out_hbm.at[pl.ds((chunk0 + j) * _CHUNK, _CHUNK)], ws[b]
                )
                if b == 0:

                    @pl.when(g >= 1)
                    def _():
                        wait_wb(p)

                    gather(j + _NBUF - 1, p)
                else:

                    @pl.when(g <= niter - 2)
                    def _():
                        wait_wb(p)
                        gather(j + _NBUF - 1, p)

            return carry

        lax.fori_loop(0, niter, outer, 0)
        for b in range(_NBUF):
            wait_wb(b)

    return pl.kernel(
        body,
        out_type=jax.ShapeDtypeStruct((n_chunks * _CHUNK, emb_dim), jnp.float32),
        mesh=mesh,
        compiler_params=pltpu.CompilerParams(use_tc_tiling_on_sc=False),
        scratch_types=(
            [pltpu.VMEM((cpw * _CHUNK,), jnp.int32)]
            + [pltpu.VMEM((_CHUNK, emb_dim), jnp.float32) for _ in range(_NBUF)]
            + [pltpu.SemaphoreType.DMA for _ in range(2 * _NBUF)]
        ),
    )


def kernel(x, table):
    b, s = x.shape
    n = b * s
    emb_dim = table.shape[1]
    idx_flat = x.reshape(n)
    out = _make_lookup(n // _CHUNK, emb_dim)(table, idx_flat)
    return out.reshape(b, s, emb_dim)
